# baseline (speedup 1.0000x reference)
import functools

import jax
import jax.numpy as jnp
from jax import lax
from jax.experimental import pallas as pl
from jax.experimental.pallas import tpu as pltpu
from jax.experimental.pallas import tpu_sc as plsc

N = 10000
D = 128
E = 320000
NC = 2
NS = 16
LANES = 16
NW = NC * NS
NPAD = 10240
ECOLS = 128
EPAD = 327680
EROWS = EPAD // ECOLS
CH = EROWS // NW
HSUB = NPAD // NS
ASUB = NPAD // NS
NBUF = 2
IDXG = 16
CH0 = 160
CH1 = 0


def _mesh():
    return plsc.VectorSubcoreMesh(core_axis_name="c", subcore_axis_name="s")


def _hist_sc(dst2d):
    @functools.partial(
        pl.kernel,
        out_type=jax.ShapeDtypeStruct((NC * NPAD,), jnp.float32),
        mesh=_mesh(),
        scratch_types=[
            pltpu.VMEM((CH, ECOLS), jnp.int32),
            pltpu.VMEM((ECOLS,), jnp.float32),
            pltpu.VMEM((HSUB,), jnp.float32),
            pltpu.VMEM_SHARED((NPAD,), jnp.float32),
        ],
    )
    def k(dst_hbm, out_hbm, idx_v, ones_v, zer_v, hist_sh):
        cid = lax.axis_index("c")
        sid = lax.axis_index("s")
        wid = sid * NC + cid

        @pl.loop(0, ECOLS, step=LANES)
        def _(i):
            ones_v[pl.ds(i, LANES)] = jnp.full((LANES,), 1.0, jnp.float32)

        @pl.loop(0, HSUB, step=LANES)
        def _(i):
            zer_v[pl.ds(i, LANES)] = jnp.zeros((LANES,), jnp.float32)

        pltpu.sync_copy(zer_v, hist_sh.at[pl.ds(sid * HSUB, HSUB)])
        plsc.subcore_barrier()

        pltpu.sync_copy(dst_hbm.at[pl.ds(wid * CH, CH)], idx_v)

        @pl.loop(0, CH)
        def _(j):
            pltpu.sync_copy(ones_v, hist_sh.at[idx_v.at[j]], add=True)

        plsc.subcore_barrier()
        pltpu.sync_copy(hist_sh.at[pl.ds(sid * HSUB, HSUB)],
                        out_hbm.at[pl.ds(cid * NPAD + sid * HSUB, HSUB)])

    return k(dst2d)


def _gather_scatter_sc(hs, src2d, dst2d):
    rows_t = pltpu.VMEM((ECOLS, D), jnp.float32)

    @functools.partial(
        pl.kernel,
        out_type=[
            jax.ShapeDtypeStruct((NPAD, D), jnp.float32),
            jax.ShapeDtypeStruct((NPAD, D), jnp.float32),
        ],
        mesh=_mesh(),
        scratch_types=[
            pltpu.VMEM((IDXG, ECOLS), jnp.int32),
            pltpu.VMEM((IDXG, ECOLS), jnp.int32),
            rows_t, rows_t,
            pltpu.SemaphoreType.DMA,
            pltpu.SemaphoreType.DMA,
            pltpu.VMEM_SHARED((NPAD, D), jnp.float32),
        ],
    )
    def k(hs_hbm, src_hbm, dst_hbm, out0_hbm, out1_hbm, src_v, dst_v,
          r0, r1, s0, s1, acc_sh):
        rows = (r0, r1)
        sems = (s0, s1)
        cid = lax.axis_index("c")
        sid = lax.axis_index("s")

        @pl.loop(0, ECOLS)
        def _(r):
            @pl.loop(0, D, step=LANES)
            def _(c):
                r0[r, pl.ds(c, LANES)] = jnp.zeros((LANES,), jnp.float32)

        @pl.loop(0, ASUB, step=ECOLS)
        def _(r):
            pltpu.sync_copy(r0, acc_sh.at[pl.ds(sid * ASUB + r, ECOLS)])

        plsc.subcore_barrier()

        def group(base):
            pltpu.sync_copy(src_hbm.at[pl.ds(base, IDXG)], src_v)
            pltpu.sync_copy(dst_hbm.at[pl.ds(base, IDXG)], dst_v)

            for b in range(NBUF):
                pltpu.async_copy(hs_hbm.at[src_v.at[b]], rows[b], sems[b])

            @pl.loop(0, IDXG, step=NBUF)
            def _(j):
                for b in range(NBUF):
                    c = j + b
                    pltpu.make_async_copy(hs_hbm.at[src_v.at[c]],
                                          rows[b], sems[b]).wait()
                    pltpu.sync_copy(rows[b], acc_sh.at[dst_v.at[c]], add=True)
                    nxt = jnp.minimum(c + NBUF, IDXG - 1)
                    pltpu.async_copy(hs_hbm.at[src_v.at[nxt]], rows[b], sems[b])

            for b in range(NBUF):
                pltpu.make_async_copy(hs_hbm.at[src_v.at[b]],
                                      rows[b], sems[b]).wait()

        @pl.when(cid == 0)
        def _():
            @pl.loop(0, CH0, step=IDXG)
            def _(g):
                group(sid * CH0 + g)

        if CH1 > 0:
            @pl.when(cid == 1)
            def _():
                @pl.loop(0, CH1, step=IDXG)
                def _(g):
                    group(NS * CH0 + sid * CH1 + g)

        plsc.subcore_barrier()

        @pl.when(cid == 0)
        def _():
            @pl.loop(0, ASUB, step=ECOLS)
            def _(r):
                pltpu.sync_copy(acc_sh.at[pl.ds(sid * ASUB + r, ECOLS)],
                                out0_hbm.at[pl.ds(sid * ASUB + r, ECOLS)])

        @pl.when(cid == 1)
        def _():
            @pl.loop(0, ASUB, step=ECOLS)
            def _(r):
                pltpu.sync_copy(acc_sh.at[pl.ds(sid * ASUB + r, ECOLS)],
                                out1_hbm.at[pl.ds(sid * ASUB + r, ECOLS)])

    return k(hs, src2d, dst2d)


def _tc_mm(x, W):
    def body(x_ref, w_ref, o_ref):
        o_ref[...] = jnp.dot(x_ref[...], w_ref[...],
                             precision=lax.Precision.HIGHEST,
                             preferred_element_type=jnp.float32)

    return pl.pallas_call(
        body, out_shape=jax.ShapeDtypeStruct((x.shape[0], W.shape[1]), jnp.float32)
    )(x, W)


def _tc_prep(hp, h1):
    def body(hp_ref, h_ref, dis_ref, hs_ref):
        dis = lax.rsqrt(hp_ref[0] + hp_ref[1] + 1.0)
        dis_ref[...] = dis
        hs_ref[...] = dis * h_ref[...]

    return pl.pallas_call(
        body,
        out_shape=[
            jax.ShapeDtypeStruct((NPAD, 1), jnp.float32),
            jax.ShapeDtypeStruct((NPAD, D), jnp.float32),
        ],
    )(hp, h1)


def _tc_mid(acc0, acc1, hs1, dis, b1, W2):
    def body(a0_ref, a1_ref, hs_ref, dis_ref, b_ref, w_ref, o_ref):
        s = a0_ref[...] + a1_ref[...] + hs_ref[...]
        h1 = jnp.maximum(dis_ref[...] * s + b_ref[...][None, :], 0.0)
        o_ref[...] = dis_ref[...] * jnp.dot(h1, w_ref[...],
                                            precision=lax.Precision.HIGHEST,
                                            preferred_element_type=jnp.float32)

    return pl.pallas_call(
        body, out_shape=jax.ShapeDtypeStruct((NPAD, D), jnp.float32)
    )(acc0, acc1, hs1, dis, b1, W2)


def _tc_final(acc0, acc1, hs2, dis, b2):
    def body(a0_ref, a1_ref, hs_ref, dis_ref, b_ref, o_ref):
        s = a0_ref[:N] + a1_ref[:N] + hs_ref[:N]
        o_ref[...] = dis_ref[:N] * s + b_ref[...][None, :]

    return pl.pallas_call(
        body, out_shape=jax.ShapeDtypeStruct((N, D), jnp.float32)
    )(acc0, acc1, hs2, dis, b2)


def kernel(x, edge_index, W1, b1, W2, b2):
    ei = edge_index.astype(jnp.int32)
    pad = jnp.full((2, EPAD - E), N, jnp.int32)
    ei = jnp.concatenate([ei, pad], axis=1)
    src2d = ei[0].reshape(EROWS, ECOLS)
    dst2d = ei[1].reshape(EROWS, ECOLS)
    xp = jnp.pad(x, ((0, NPAD - N), (0, 0)))

    h1 = _tc_mm(xp, W1)
    hist = _hist_sc(dst2d)
    hp = hist.reshape(NC, NPAD, 1)

    dis, hs1 = _tc_prep(hp, h1)
    acc1a, acc1b = _gather_scatter_sc(hs1, src2d, dst2d)
    hs2 = _tc_mid(acc1a, acc1b, hs1, dis, b1, W2)
    acc2a, acc2b = _gather_scatter_sc(hs2, src2d, dst2d)
    return _tc_final(acc2a, acc2b, hs2, dis, b2)


# speedup vs baseline: 1.5052x; 1.5052x over previous
import functools

import jax
import jax.numpy as jnp
from jax import lax
from jax.experimental import pallas as pl
from jax.experimental.pallas import tpu as pltpu
from jax.experimental.pallas import tpu_sc as plsc

N = 10000
D = 128
E = 320000
NC = 2
NS = 16
LANES = 16
NW = NC * NS
NPAD = 10240
ECOLS = 128
EPAD = 327680
EROWS = EPAD // ECOLS
CH = EROWS // NW
HSUB = NPAD // NS
ASUB = NPAD // NS
NBUF = 2
IDXG = 16
CH0 = 144
CH1 = 16


def _mesh():
    return plsc.VectorSubcoreMesh(core_axis_name="c", subcore_axis_name="s")


def _hist_sc(dst2d):
    @functools.partial(
        pl.kernel,
        out_type=jax.ShapeDtypeStruct((NC * NPAD,), jnp.float32),
        mesh=_mesh(),
        scratch_types=[
            pltpu.VMEM((CH, ECOLS), jnp.int32),
            pltpu.VMEM((ECOLS,), jnp.float32),
            pltpu.VMEM((HSUB,), jnp.float32),
            pltpu.VMEM_SHARED((NPAD,), jnp.float32),
        ],
    )
    def k(dst_hbm, out_hbm, idx_v, ones_v, zer_v, hist_sh):
        cid = lax.axis_index("c")
        sid = lax.axis_index("s")
        wid = sid * NC + cid

        @pl.loop(0, ECOLS, step=LANES)
        def _(i):
            ones_v[pl.ds(i, LANES)] = jnp.full((LANES,), 1.0, jnp.float32)

        @pl.loop(0, HSUB, step=LANES)
        def _(i):
            zer_v[pl.ds(i, LANES)] = jnp.zeros((LANES,), jnp.float32)

        pltpu.sync_copy(zer_v, hist_sh.at[pl.ds(sid * HSUB, HSUB)])
        plsc.subcore_barrier()

        pltpu.sync_copy(dst_hbm.at[pl.ds(wid * CH, CH)], idx_v)

        @pl.loop(0, CH)
        def _(j):
            pltpu.sync_copy(ones_v, hist_sh.at[idx_v.at[j]], add=True)

        plsc.subcore_barrier()
        pltpu.sync_copy(hist_sh.at[pl.ds(sid * HSUB, HSUB)],
                        out_hbm.at[pl.ds(cid * NPAD + sid * HSUB, HSUB)])

    return k(dst2d)


def _gather_scatter_sc(hs, src2d, dst2d):
    rows_t = pltpu.VMEM((ECOLS, D), jnp.float32)

    @functools.partial(
        pl.kernel,
        out_type=[
            jax.ShapeDtypeStruct((NPAD, D), jnp.float32),
            jax.ShapeDtypeStruct((NPAD, D), jnp.float32),
        ],
        mesh=_mesh(),
        scratch_types=[
            pltpu.VMEM((IDXG, ECOLS), jnp.int32),
            pltpu.VMEM((IDXG, ECOLS), jnp.int32),
            rows_t, rows_t,
            pltpu.SemaphoreType.DMA,
            pltpu.SemaphoreType.DMA,
            pltpu.VMEM_SHARED((NPAD, D), jnp.float32),
        ],
    )
    def k(hs_hbm, src_hbm, dst_hbm, out0_hbm, out1_hbm, src_v, dst_v,
          r0, r1, s0, s1, acc_sh):
        rows = (r0, r1)
        sems = (s0, s1)
        cid = lax.axis_index("c")
        sid = lax.axis_index("s")

        @pl.loop(0, ECOLS)
        def _(r):
            @pl.loop(0, D, step=LANES)
            def _(c):
                r0[r, pl.ds(c, LANES)] = jnp.zeros((LANES,), jnp.float32)

        @pl.loop(0, ASUB, step=ECOLS)
        def _(r):
            pltpu.sync_copy(r0, acc_sh.at[pl.ds(sid * ASUB + r, ECOLS)])

        plsc.subcore_barrier()

        def group(base):
            pltpu.sync_copy(src_hbm.at[pl.ds(base, IDXG)], src_v)
            pltpu.sync_copy(dst_hbm.at[pl.ds(base, IDXG)], dst_v)

            for b in range(NBUF):
                pltpu.async_copy(hs_hbm.at[src_v.at[b]], rows[b], sems[b])

            @pl.loop(0, IDXG, step=NBUF)
            def _(j):
                for b in range(NBUF):
                    c = j + b
                    pltpu.make_async_copy(hs_hbm.at[src_v.at[c]],
                                          rows[b], sems[b]).wait()
                    pltpu.sync_copy(rows[b], acc_sh.at[dst_v.at[c]], add=True)
                    nxt = jnp.minimum(c + NBUF, IDXG - 1)
                    pltpu.async_copy(hs_hbm.at[src_v.at[nxt]], rows[b], sems[b])

            for b in range(NBUF):
                pltpu.make_async_copy(hs_hbm.at[src_v.at[b]],
                                      rows[b], sems[b]).wait()

        @pl.when(cid == 0)
        def _():
            @pl.loop(0, CH0, step=IDXG)
            def _(g):
                group(sid * CH0 + g)

        if CH1 > 0:
            @pl.when(cid == 1)
            def _():
                @pl.loop(0, CH1, step=IDXG)
                def _(g):
                    group(NS * CH0 + sid * CH1 + g)

        plsc.subcore_barrier()

        @pl.when(cid == 0)
        def _():
            @pl.loop(0, ASUB, step=ECOLS)
            def _(r):
                pltpu.sync_copy(acc_sh.at[pl.ds(sid * ASUB + r, ECOLS)],
                                out0_hbm.at[pl.ds(sid * ASUB + r, ECOLS)])

        @pl.when(cid == 1)
        def _():
            @pl.loop(0, ASUB, step=ECOLS)
            def _(r):
                pltpu.sync_copy(acc_sh.at[pl.ds(sid * ASUB + r, ECOLS)],
                                out1_hbm.at[pl.ds(sid * ASUB + r, ECOLS)])

    return k(hs, src2d, dst2d)


def _tc_mm(x, W):
    def body(x_ref, w_ref, o_ref):
        o_ref[...] = jnp.dot(x_ref[...], w_ref[...],
                             precision=lax.Precision.HIGHEST,
                             preferred_element_type=jnp.float32)

    return pl.pallas_call(
        body, out_shape=jax.ShapeDtypeStruct((x.shape[0], W.shape[1]), jnp.float32)
    )(x, W)


def _tc_prep(hp, h1):
    def body(hp_ref, h_ref, dis_ref, hs_ref):
        dis = lax.rsqrt(hp_ref[0] + hp_ref[1] + 1.0)
        dis_ref[...] = dis
        hs_ref[...] = dis * h_ref[...]

    return pl.pallas_call(
        body,
        out_shape=[
            jax.ShapeDtypeStruct((NPAD, 1), jnp.float32),
            jax.ShapeDtypeStruct((NPAD, D), jnp.float32),
        ],
    )(hp, h1)


def _tc_mid(acc0, acc1, hs1, dis, b1, W2):
    def body(a0_ref, a1_ref, hs_ref, dis_ref, b_ref, w_ref, o_ref):
        s = a0_ref[...] + a1_ref[...] + hs_ref[...]
        h1 = jnp.maximum(dis_ref[...] * s + b_ref[...][None, :], 0.0)
        o_ref[...] = dis_ref[...] * jnp.dot(h1, w_ref[...],
                                            precision=lax.Precision.HIGHEST,
                                            preferred_element_type=jnp.float32)

    return pl.pallas_call(
        body, out_shape=jax.ShapeDtypeStruct((NPAD, D), jnp.float32)
    )(acc0, acc1, hs1, dis, b1, W2)


def _tc_final(acc0, acc1, hs2, dis, b2):
    def body(a0_ref, a1_ref, hs_ref, dis_ref, b_ref, o_ref):
        s = a0_ref[:N] + a1_ref[:N] + hs_ref[:N]
        o_ref[...] = dis_ref[:N] * s + b_ref[...][None, :]

    return pl.pallas_call(
        body, out_shape=jax.ShapeDtypeStruct((N, D), jnp.float32)
    )(acc0, acc1, hs2, dis, b2)


def kernel(x, edge_index, W1, b1, W2, b2):
    ei = edge_index.astype(jnp.int32)
    pad = jnp.full((2, EPAD - E), N, jnp.int32)
    ei = jnp.concatenate([ei, pad], axis=1)
    src2d = ei[0].reshape(EROWS, ECOLS)
    dst2d = ei[1].reshape(EROWS, ECOLS)
    xp = jnp.pad(x, ((0, NPAD - N), (0, 0)))

    h1 = _tc_mm(xp, W1)
    hist = _hist_sc(dst2d)
    hp = hist.reshape(NC, NPAD, 1)

    dis, hs1 = _tc_prep(hp, h1)
    acc1a, acc1b = _gather_scatter_sc(hs1, src2d, dst2d)
    hs2 = _tc_mid(acc1a, acc1b, hs1, dis, b1, W2)
    acc2a, acc2b = _gather_scatter_sc(hs2, src2d, dst2d)
    return _tc_final(acc2a, acc2b, hs2, dis, b2)
